# baseline (speedup 1.0000x reference)
import jax
import jax.numpy as jnp
from jax import lax
from jax.experimental import pallas as pl
from jax.experimental.pallas import tpu as pltpu
from jax.experimental.pallas import tpu_sc as plsc

N = 50000
E = 800000
F_IN = 128
H = 64
G = 64
NUM_FAM = 100
CPF = 3

NC = 2
NS = 16
RB = 2048
NRB = 25
PR = RB // 4
N_PAD = NRB * RB
N_ACC = 53248
NBA = N_ACC // RB
ROWS_PER_TILE = N_ACC // NS
ZROWS = 64
E_PAD = 819200
EROWS_MAIN = E_PAD // NS // 128
BLK = 4
NBLK_MAIN = EROWS_MAIN // BLK
EROWS_DEG = E_PAD // NC // NS // 128
NBLK_DEG = EROWS_DEG // BLK


def _deg_body(dst_hbm, out_hbm, dbufA, dbufB, ones_v, zbuf, dacc,
              sem0, sem1, sem2, sem3):
    sems = (sem0, sem1, sem2, sem3)
    c = lax.axis_index("c")
    s = lax.axis_index("s")

    def fill(i, _):
        zbuf[i, 0:16] = jnp.zeros((16,), jnp.float32)
        zbuf[i, 16:32] = jnp.zeros((16,), jnp.float32)
        ones_v[i, 0:16] = jnp.full((16,), 1.0, jnp.float32)
        ones_v[i, 16:32] = jnp.full((16,), 1.0, jnp.float32)
        return 0

    lax.fori_loop(0, 128, fill, 0)

    def zero(t, _):
        pltpu.sync_copy(zbuf, dacc.at[pl.ds(s * ROWS_PER_TILE + t * 128, 128)])
        return 0

    lax.fori_loop(0, ROWS_PER_TILE // 128, zero, 0)
    plsc.subcore_barrier()

    row0 = (c * NS + s) * EROWS_DEG

    def stage(b, db):
        pltpu.sync_copy(dst_hbm.at[pl.ds(row0 + b * BLK, BLK)], db)

    def fire(j, db):
        pltpu.async_copy(ones_v, dacc.at[db.at[j]], sems[j], add=True)

    def wait(j, db):
        pltpu.make_async_copy(ones_v, dacc.at[db.at[j]], sems[j]).wait()

    stage(0, dbufA)

    def body(t, _):
        bA = 2 * t
        last = t == NBLK_DEG // 2 - 1
        stage(bA + 1, dbufB)
        for j in range(BLK):
            fire(j, dbufA)
        for j in range(BLK):
            wait(j, dbufA)

        @pl.when(jnp.logical_not(last))
        def _():
            stage(bA + 2, dbufA)

        for j in range(BLK):
            fire(j, dbufB)
        for j in range(BLK):
            wait(j, dbufB)
        return 0

    lax.fori_loop(0, NBLK_DEG // 2, body, 0)
    plsc.subcore_barrier()
    pltpu.sync_copy(dacc.at[pl.ds(s * ROWS_PER_TILE, ROWS_PER_TILE)],
                    out_hbm.at[pl.ds((c * NS + s) * ROWS_PER_TILE,
                                     ROWS_PER_TILE)])


def _scat_body(src_hbm, dst_hbm, tab_hbm, out_hbm, sbufA, dbufA, sbufB,
               dbufB, msgs, zbuf, acc, sem0, sem1, sem2, sem3,
               ssem0, ssem1, ssem2, ssem3):
    sems = (sem0, sem1, sem2, sem3)
    ssems = (ssem0, ssem1, ssem2, ssem3)
    c = lax.axis_index("c")
    s = lax.axis_index("s")
    srow0 = (c * NS + s) * EROWS_MAIN
    drow0 = s * EROWS_MAIN

    def stage(b, sb, db):
        pltpu.sync_copy(src_hbm.at[pl.ds(srow0 + b * BLK, BLK)], sb)
        pltpu.sync_copy(dst_hbm.at[pl.ds(drow0 + b * BLK, BLK)], db)

    def fire(j, sb):
        pltpu.async_copy(tab_hbm.at[sb.at[j]],
                         msgs.at[pl.ds(j * 128, 128)], sems[j])

    def wait(j, sb):
        pltpu.make_async_copy(tab_hbm.at[sb.at[j]],
                              msgs.at[pl.ds(j * 128, 128)], sems[j]).wait()

    def scat_fire(j, db):
        pltpu.async_copy(msgs.at[pl.ds(j * 128, 128)], acc.at[db.at[j]],
                         ssems[j], add=True)

    def scat_wait(j, db):
        pltpu.make_async_copy(msgs.at[pl.ds(j * 128, 128)],
                              acc.at[db.at[j]], ssems[j]).wait()

    stage(0, sbufA, dbufA)
    for j in range(BLK):
        fire(j, sbufA)

    def fill(i, _):
        zbuf[i, 0:16] = jnp.zeros((16,), jnp.float32)
        zbuf[i, 16:32] = jnp.zeros((16,), jnp.float32)
        return 0

    lax.fori_loop(0, ZROWS, fill, 0)

    def zero(t, _):
        pltpu.sync_copy(zbuf,
                        acc.at[pl.ds(s * ROWS_PER_TILE + t * ZROWS, ZROWS)])
        return 0

    lax.fori_loop(0, ROWS_PER_TILE // ZROWS, zero, 0)
    plsc.subcore_barrier()

    def outer(t, _):
        bA = 2 * t
        last = t == NBLK_MAIN // 2 - 1
        stage(bA + 1, sbufB, dbufB)
        for j in range(BLK):
            wait(j, sbufA)
            scat_fire(j, dbufA)
        for j in range(BLK):
            scat_wait(j, dbufA)
            fire(j, sbufB)

        @pl.when(jnp.logical_not(last))
        def _():
            stage(bA + 2, sbufA, dbufA)

        for j in range(BLK):
            wait(j, sbufB)
            scat_fire(j, dbufB)
        for j in range(BLK):
            scat_wait(j, dbufB)

            @pl.when(jnp.logical_not(last))
            def _(j=j):
                fire(j, sbufA)
        return 0

    lax.fori_loop(0, NBLK_MAIN // 2, outer, 0)
    plsc.subcore_barrier()
    pltpu.sync_copy(acc.at[pl.ds(s * ROWS_PER_TILE, ROWS_PER_TILE)],
                    out_hbm.at[pl.ds((c * NS + s) * ROWS_PER_TILE,
                                     ROWS_PER_TILE)])


_SC_CACHE = []


def _sc_calls():
    if _SC_CACHE:
        return _SC_CACHE[0]
    mesh = plsc.VectorSubcoreMesh(core_axis_name="c", subcore_axis_name="s",
                                  num_cores=NC, num_subcores=NS)
    params = pltpu.CompilerParams(use_tc_tiling_on_sc=False)
    deg = pl.kernel(
        _deg_body,
        out_type=jax.ShapeDtypeStruct((NC * N_ACC, 32), jnp.float32),
        mesh=mesh,
        compiler_params=params,
        scratch_types=[
            pltpu.VMEM((BLK, 128), jnp.int32),
            pltpu.VMEM((BLK, 128), jnp.int32),
            pltpu.VMEM((128, 32), jnp.float32),
            pltpu.VMEM((128, 32), jnp.float32),
            pltpu.VMEM_SHARED((N_ACC, 32), jnp.float32),
            pltpu.SemaphoreType.DMA,
            pltpu.SemaphoreType.DMA,
            pltpu.SemaphoreType.DMA,
            pltpu.SemaphoreType.DMA,
        ],
    )
    scat = pl.kernel(
        _scat_body,
        out_type=jax.ShapeDtypeStruct((NC * N_ACC, 32), jnp.float32),
        mesh=mesh,
        compiler_params=params,
        scratch_types=[
            pltpu.VMEM((BLK, 128), jnp.int32),
            pltpu.VMEM((BLK, 128), jnp.int32),
            pltpu.VMEM((BLK, 128), jnp.int32),
            pltpu.VMEM((BLK, 128), jnp.int32),
            pltpu.VMEM((BLK * 128, 32), jnp.float32),
            pltpu.VMEM((ZROWS, 32), jnp.float32),
            pltpu.VMEM_SHARED((N_ACC, 32), jnp.float32),
            pltpu.SemaphoreType.DMA,
            pltpu.SemaphoreType.DMA,
            pltpu.SemaphoreType.DMA,
            pltpu.SemaphoreType.DMA,
            pltpu.SemaphoreType.DMA,
            pltpu.SemaphoreType.DMA,
            pltpu.SemaphoreType.DMA,
            pltpu.SemaphoreType.DMA,
        ],
    )
    _SC_CACHE.append((deg, scat))
    return _SC_CACHE[0]


def _deg_call(dst_r):
    return _sc_calls()[0](dst_r)


def _scat_call(src2, dst_r, tab):
    return _sc_calls()[1](src2, dst_r, tab)


def _dinv_j(dlo_ref, dhi_ref, j):
    d = (dlo_ref[:, 32 * j:32 * j + 1] + dhi_ref[:, 32 * j:32 * j + 1] + 1.0)
    return lax.rsqrt(d)


def _d1_body(x_ref, w_ref, dlo_ref, dhi_ref, out_ref, dcache):
    h0 = pl.program_id(0)
    i = pl.program_id(1)
    for j in range(4):
        xj = x_ref[pl.ds(j * PR, PR), :]
        h = jnp.dot(xj, w_ref[0], preferred_element_type=jnp.float32)

        @pl.when(h0 == 0)
        def _(j=j):
            dcache[pl.ds(i * PR, PR), j:j + 1] = _dinv_j(dlo_ref, dhi_ref, j)
        out_ref[:, 32 * j:32 * (j + 1)] = (
            h * dcache[pl.ds(i * PR, PR), j:j + 1])


_d1_call = pl.pallas_call(
    _d1_body,
    grid=(2, NRB),
    in_specs=[
        pl.BlockSpec((RB, F_IN), lambda h, i: (i, 0)),
        pl.BlockSpec((1, F_IN, 32), lambda h, i: (h, 0, 0)),
        pl.BlockSpec((PR, 128), lambda h, i: (jnp.where(h == 0, i, 0), 0)),
        pl.BlockSpec((PR, 128),
                     lambda h, i: (NBA + jnp.where(h == 0, i, 0), 0)),
    ],
    out_specs=pl.BlockSpec((PR, 128), lambda h, i: (h * NRB + i, 0)),
    out_shape=jax.ShapeDtypeStruct((2 * N_PAD // 4, 128), jnp.float32),
    scratch_shapes=[pltpu.VMEM((N_PAD // 4, 8), jnp.float32)],
)


def _zpart_j(alo, ahi, hlo, hhi, dinv, b_ref, j):
    lo = (alo[:, 32 * j:32 * (j + 1)] + hlo[:, 32 * j:32 * (j + 1)])
    hi = (ahi[:, 32 * j:32 * (j + 1)] + hhi[:, 32 * j:32 * (j + 1)])
    return jnp.concatenate([lo, hi], axis=1) * dinv + b_ref[...]


def _stat_mask(k, j):
    p = lax.broadcasted_iota(jnp.int32, (PR, 1), 0)
    nid = (k * PR + p) * 4 + j
    return nid < N


def _phase1(k, acc_lo, acc_hi, hp_lo, hp_hi, dlo, dhi, b_ref, zbuf, sacc,
            dcache):
    alo, ahi = acc_lo[...], acc_hi[...]
    hlo, hhi = hp_lo[...], hp_hi[...]
    tot = None
    for j in range(4):
        dinv = _dinv_j(dlo, dhi, j)
        dcache[pl.ds(k * PR, PR), j:j + 1] = dinv
        z = _zpart_j(alo, ahi, hlo, hhi, dinv, b_ref, j)
        zbuf[pl.ds(k * PR, PR), 64 * j:64 * (j + 1)] = z
        zm = jnp.where(_stat_mask(k, j), z, 0.0)
        part = jnp.concatenate([jnp.sum(zm, axis=0, keepdims=True),
                                jnp.sum(zm * zm, axis=0, keepdims=True)],
                               axis=0)
        tot = part if tot is None else tot + part

    @pl.when(k == 0)
    def _():
        sacc[...] = tot

    @pl.when(k > 0)
    def _():
        sacc[...] = sacc[...] + tot


def _bn_coefs(sacc, g_ref, be_ref):
    m = sacc[0:1, :] * (1.0 / N)
    var = sacc[1:2, :] * (1.0 / N) - m * m
    sc = g_ref[...] * lax.rsqrt(var + 1e-5)
    sh = be_ref[...] - m * sc
    return sc, sh


def _m_body(acc_lo, acc_hi, hp_lo, hp_hi, dlo, dhi, b_ref, g_ref, be_ref,
            w_ref, out_ref, zbuf, sacc, dcache):
    k = pl.program_id(0)

    @pl.when(k < NRB)
    def _():
        _phase1(k, acc_lo, acc_hi, hp_lo, hp_hi, dlo, dhi, b_ref, zbuf, sacc,
                dcache)

    @pl.when(k >= NRB)
    def _():
        i2 = lax.rem(k - NRB, NRB)
        sc, sh = _bn_coefs(sacc[...], g_ref, be_ref)
        for j in range(4):
            zj = zbuf[pl.ds(i2 * PR, PR), 64 * j:64 * (j + 1)]
            act = jnp.maximum(zj * sc + sh, 0.0)
            h = jnp.dot(act, w_ref[0], preferred_element_type=jnp.float32)
            out_ref[:, 32 * j:32 * (j + 1)] = (
                h * dcache[pl.ds(i2 * PR, PR), j:j + 1])


def _zk(k):
    return jnp.where(k < NRB, k, lax.rem(k - NRB, NRB))


_m_call = pl.pallas_call(
    _m_body,
    grid=(3 * NRB,),
    in_specs=[
        pl.BlockSpec((PR, 128), lambda k: (jnp.where(k < NRB, k, 0), 0)),
        pl.BlockSpec((PR, 128),
                     lambda k: (NBA + jnp.where(k < NRB, k, 0), 0)),
        pl.BlockSpec((PR, 128), lambda k: (jnp.where(k < NRB, k, 0), 0)),
        pl.BlockSpec((PR, 128),
                     lambda k: (NRB + jnp.where(k < NRB, k, 0), 0)),
        pl.BlockSpec((PR, 128), lambda k: (jnp.where(k < NRB, k, 0), 0)),
        pl.BlockSpec((PR, 128),
                     lambda k: (NBA + jnp.where(k < NRB, k, 0), 0)),
        pl.BlockSpec((1, H), lambda k: (0, 0)),
        pl.BlockSpec((1, H), lambda k: (0, 0)),
        pl.BlockSpec((1, H), lambda k: (0, 0)),
        pl.BlockSpec((1, H, 32),
                     lambda k: (jnp.where(k < NRB, 0, (k - NRB) // NRB),
                                0, 0)),
    ],
    out_specs=pl.BlockSpec(
        (PR, 128),
        lambda k: (jnp.where(k < NRB, 0, (k - NRB) // NRB) * NRB
                   + jnp.where(k < NRB, 0, lax.rem(k - NRB, NRB)), 0)),
    out_shape=jax.ShapeDtypeStruct((2 * N_PAD // 4, 128), jnp.float32),
    scratch_shapes=[pltpu.VMEM((N_PAD // 4, 256), jnp.float32),
                    pltpu.VMEM((2, H), jnp.float32),
                    pltpu.VMEM((N_PAD // 4, 8), jnp.float32)],
)


def _p_body(acc_lo, acc_hi, hp_lo, hp_hi, dlo, dhi, b_ref, g_ref, be_ref,
            bat_ref, gc_ref, mal_ref, t_ref, out_ref, zbuf, sacc, psum, pcnt,
            dcache):
    k = pl.program_id(0)

    @pl.when(k < NRB)
    def _():
        _phase1(k, acc_lo, acc_hi, hp_lo, hp_hi, dlo, dhi, b_ref, zbuf, sacc,
                dcache)

    @pl.when(k >= NRB)
    def _():
        i2 = k - NRB
        sc, sh = _bn_coefs(sacc[...], g_ref, be_ref)
        ps = None
        pc = None
        gids = lax.broadcasted_iota(jnp.int32, (PR, G), 1)
        for j in range(4):
            zj = zbuf[pl.ds(i2 * PR, PR), 64 * j:64 * (j + 1)]
            nz = zj * sc + sh
            bat = bat_ref[pl.ds(j * PR, PR), :]
            oneh = (bat == gids).astype(jnp.float32)
            psj = lax.dot_general(oneh, nz, (((0,), (0,)), ((), ())),
                                  preferred_element_type=jnp.float32)
            pcj = lax.dot_general(oneh, jnp.ones((PR, 8), jnp.float32),
                                  (((0,), (0,)), ((), ())),
                                  preferred_element_type=jnp.float32)
            ps = psj if ps is None else ps + psj
            pc = pcj if pc is None else pc + pcj

        @pl.when(i2 == 0)
        def _():
            psum[...] = ps
            pcnt[...] = pc

        @pl.when(i2 > 0)
        def _():
            psum[...] = psum[...] + ps
            pcnt[...] = pcnt[...] + pc

    @pl.when(k == 2 * NRB - 1)
    def _():
        emb = psum[...] / jnp.maximum(pcnt[...][:, 0:1], 1.0)
        e2 = jnp.sum(emb * emb, axis=1, keepdims=True)
        gcT = gc_ref[...]
        gd = (e2 + jnp.sum(gcT * gcT, axis=0, keepdims=True)
              - 2.0 * jnp.dot(emb, gcT, preferred_element_type=jnp.float32))
        ming = jnp.min(gd, axis=1, keepdims=True)
        mm = None
        for kk in range(CPF):
            mk = mal_ref[kk]
            mdk = (e2 + jnp.sum(mk * mk, axis=0, keepdims=True)
                   - 2.0 * jnp.dot(emb, mk,
                                   preferred_element_type=jnp.float32))
            mm = mdk if mm is None else jnp.minimum(mm, mdk)
        tv = t_ref[0, 0]
        out_ref[...] = jnp.concatenate([-ming, -mm], axis=1) / tv


_p_call = pl.pallas_call(
    _p_body,
    grid=(2 * NRB,),
    in_specs=[
        pl.BlockSpec((PR, 128), lambda k: (jnp.where(k < NRB, k, 0), 0)),
        pl.BlockSpec((PR, 128),
                     lambda k: (NBA + jnp.where(k < NRB, k, 0), 0)),
        pl.BlockSpec((PR, 128), lambda k: (jnp.where(k < NRB, k, 0), 0)),
        pl.BlockSpec((PR, 128),
                     lambda k: (NRB + jnp.where(k < NRB, k, 0), 0)),
        pl.BlockSpec((PR, 128), lambda k: (jnp.where(k < NRB, k, 0), 0)),
        pl.BlockSpec((PR, 128),
                     lambda k: (NBA + jnp.where(k < NRB, k, 0), 0)),
        pl.BlockSpec((1, H), lambda k: (0, 0)),
        pl.BlockSpec((1, H), lambda k: (0, 0)),
        pl.BlockSpec((1, H), lambda k: (0, 0)),
        pl.BlockSpec((RB, 1), lambda k: (jnp.where(k < NRB, 0, k - NRB), 0)),
        pl.BlockSpec((H, 8), lambda k: (0, 0)),
        pl.BlockSpec((CPF, H, NUM_FAM), lambda k: (0, 0, 0)),
        pl.BlockSpec((1, 1), lambda k: (0, 0)),
    ],
    out_specs=pl.BlockSpec((G, 1 + NUM_FAM), lambda k: (0, 0)),
    out_shape=jax.ShapeDtypeStruct((G, 1 + NUM_FAM), jnp.float32),
    scratch_shapes=[pltpu.VMEM((N_PAD // 4, 256), jnp.float32),
                    pltpu.VMEM((2, H), jnp.float32),
                    pltpu.VMEM((G, H), jnp.float32),
                    pltpu.VMEM((G, 8), jnp.float32),
                    pltpu.VMEM((N_PAD // 4, 8), jnp.float32)],
)


def kernel(x, edge_index, batch, W1, b1, W2, b2, W3, b3, g1, be1, g2, be2,
           g3, be3, mal_c, good_c, temp):
    f32 = jnp.float32
    src = edge_index[0]
    dst = edge_index[1]
    pad = jnp.arange(E_PAD - E, dtype=jnp.int32)
    src_p = jnp.concatenate([src, pad % N])
    dst_p = jnp.concatenate([dst, N + pad % (N_ACC - N)])
    src2 = jnp.concatenate([src_p, src_p + N_PAD]).reshape(
        2 * E_PAD // 128, 128)
    dst_r = dst_p.reshape(E_PAD // 128, 128)

    x_pad = jnp.concatenate([x, jnp.zeros((N_PAD - N, F_IN), f32)])
    x_perm = x_pad.reshape(NRB, PR, 4, F_IN).transpose(0, 2, 1, 3)
    x_perm = x_perm.reshape(N_PAD, F_IN)
    bat_pad = jnp.concatenate(
        [batch, jnp.full((N_PAD - N,), G, jnp.int32)])
    bat_perm = bat_pad.reshape(NRB, PR, 4).transpose(0, 2, 1)
    bat_perm = bat_perm.reshape(N_PAD, 1)

    w1s = W1.reshape(F_IN, 2, 32).transpose(1, 0, 2)
    w2s = W2.reshape(H, 2, 32).transpose(1, 0, 2)
    w3s = W3.reshape(H, 2, 32).transpose(1, 0, 2)

    degp = _deg_call(dst_r).reshape(NC * N_ACC // 4, 128)

    hp1 = _d1_call(x_perm, w1s, degp, degp)
    acc1 = _scat_call(src2, dst_r, hp1.reshape(2 * N_PAD, 32))
    acc1 = acc1.reshape(NC * N_ACC // 4, 128)
    hp2 = _m_call(acc1, acc1, hp1, hp1, degp, degp, b1.reshape(1, H),
                  g1.reshape(1, H), be1.reshape(1, H), w2s)
    acc2 = _scat_call(src2, dst_r, hp2.reshape(2 * N_PAD, 32))
    acc2 = acc2.reshape(NC * N_ACC // 4, 128)
    hp3 = _m_call(acc2, acc2, hp2, hp2, degp, degp, b2.reshape(1, H),
                  g2.reshape(1, H), be2.reshape(1, H), w3s)
    acc3 = _scat_call(src2, dst_r, hp3.reshape(2 * N_PAD, 32))
    acc3 = acc3.reshape(NC * N_ACC // 4, 128)

    good_pad = jnp.concatenate([good_c, jnp.full((3, H), 1e4, f32)], axis=0)
    mal3 = mal_c.reshape(NUM_FAM, CPF, H).transpose(1, 2, 0)
    logits = _p_call(acc3, acc3, hp3, hp3, degp, degp, b3.reshape(1, H),
                     g3.reshape(1, H), be3.reshape(1, H), bat_perm,
                     good_pad.T, mal3, temp.reshape(1, 1))
    return logits


# speedup vs baseline: 1.0134x; 1.0134x over previous
import jax
import jax.numpy as jnp
from jax import lax
from jax.experimental import pallas as pl
from jax.experimental.pallas import tpu as pltpu
from jax.experimental.pallas import tpu_sc as plsc

N = 50000
E = 800000
F_IN = 128
H = 64
G = 64
NUM_FAM = 100
CPF = 3

NC = 2
NS = 16
RB = 2048
NRB = 25
PR = RB // 4
N_PAD = NRB * RB
N_ACC = 53248
NBA = N_ACC // RB
ROWS_PER_TILE = N_ACC // NS
ZROWS = 64
E_PAD = 819200
EROWS_MAIN = E_PAD // NS // 128
BLK = 4
NBLK_MAIN = EROWS_MAIN // BLK
EROWS_DEG = E_PAD // NC // NS // 128
NBLK_DEG = EROWS_DEG // BLK


def _deg_body(dst_hbm, out_hbm, dbufA, dbufB, ones_v, zbuf, dacc,
              sem0, sem1, sem2, sem3):
    sems = (sem0, sem1, sem2, sem3)
    c = lax.axis_index("c")
    s = lax.axis_index("s")

    def fill(i, _):
        zbuf[i, 0:16] = jnp.zeros((16,), jnp.float32)
        zbuf[i, 16:32] = jnp.zeros((16,), jnp.float32)
        ones_v[i, 0:16] = jnp.full((16,), 1.0, jnp.float32)
        ones_v[i, 16:32] = jnp.full((16,), 1.0, jnp.float32)
        return 0

    lax.fori_loop(0, 128, fill, 0)

    def zero(t, _):
        pltpu.sync_copy(zbuf, dacc.at[pl.ds(s * ROWS_PER_TILE + t * 128, 128)])
        return 0

    lax.fori_loop(0, ROWS_PER_TILE // 128, zero, 0)
    plsc.subcore_barrier()

    row0 = (c * NS + s) * EROWS_DEG

    def stage(b, db):
        pltpu.sync_copy(dst_hbm.at[pl.ds(row0 + b * BLK, BLK)], db)

    def fire(j, db):
        pltpu.async_copy(ones_v, dacc.at[db.at[j]], sems[j], add=True)

    def wait(j, db):
        pltpu.make_async_copy(ones_v, dacc.at[db.at[j]], sems[j]).wait()

    stage(0, dbufA)

    def body(t, _):
        bA = 2 * t
        last = t == NBLK_DEG // 2 - 1
        stage(bA + 1, dbufB)
        for j in range(BLK):
            fire(j, dbufA)
        for j in range(BLK):
            wait(j, dbufA)

        @pl.when(jnp.logical_not(last))
        def _():
            stage(bA + 2, dbufA)

        for j in range(BLK):
            fire(j, dbufB)
        for j in range(BLK):
            wait(j, dbufB)
        return 0

    lax.fori_loop(0, NBLK_DEG // 2, body, 0)
    plsc.subcore_barrier()
    pltpu.sync_copy(dacc.at[pl.ds(s * ROWS_PER_TILE, ROWS_PER_TILE)],
                    out_hbm.at[pl.ds((c * NS + s) * ROWS_PER_TILE,
                                     ROWS_PER_TILE)])


def _scat_body(src_hbm, dst_hbm, tab_hbm, out_hbm, sbufA, dbufA, sbufB,
               dbufB, msgs, zbuf, acc, sem0, sem1, sem2, sem3,
               ssem0, ssem1, ssem2, ssem3):
    sems = (sem0, sem1, sem2, sem3)
    ssems = (ssem0, ssem1, ssem2, ssem3)
    c = lax.axis_index("c")
    s = lax.axis_index("s")
    srow0 = (c * NS + s) * EROWS_MAIN
    drow0 = s * EROWS_MAIN

    def stage(b, sb, db):
        pltpu.sync_copy(src_hbm.at[pl.ds(srow0 + b * BLK, BLK)], sb)
        pltpu.sync_copy(dst_hbm.at[pl.ds(drow0 + b * BLK, BLK)], db)

    def fire(j, sb):
        pltpu.async_copy(tab_hbm.at[sb.at[j]],
                         msgs.at[pl.ds(j * 128, 128)], sems[j])

    def wait(j, sb):
        pltpu.make_async_copy(tab_hbm.at[sb.at[j]],
                              msgs.at[pl.ds(j * 128, 128)], sems[j]).wait()

    def scat_fire(j, db):
        pltpu.async_copy(msgs.at[pl.ds(j * 128, 128)], acc.at[db.at[j]],
                         ssems[j], add=True)

    def scat_wait(j, db):
        pltpu.make_async_copy(msgs.at[pl.ds(j * 128, 128)],
                              acc.at[db.at[j]], ssems[j]).wait()

    stage(0, sbufA, dbufA)
    for j in range(BLK):
        fire(j, sbufA)

    def fill(i, _):
        zbuf[i, 0:16] = jnp.zeros((16,), jnp.float32)
        zbuf[i, 16:32] = jnp.zeros((16,), jnp.float32)
        return 0

    lax.fori_loop(0, ZROWS, fill, 0)

    def zero(t, _):
        pltpu.sync_copy(zbuf,
                        acc.at[pl.ds(s * ROWS_PER_TILE + t * ZROWS, ZROWS)])
        return 0

    lax.fori_loop(0, ROWS_PER_TILE // ZROWS, zero, 0)
    plsc.subcore_barrier()

    def outer(t, _):
        bA = 2 * t
        last = t == NBLK_MAIN // 2 - 1
        stage(bA + 1, sbufB, dbufB)
        for j in range(BLK):
            wait(j, sbufA)
            scat_fire(j, dbufA)
        for j in range(BLK):
            scat_wait(j, dbufA)
            fire(j, sbufB)

        @pl.when(jnp.logical_not(last))
        def _():
            stage(bA + 2, sbufA, dbufA)

        for j in range(BLK):
            wait(j, sbufB)
            scat_fire(j, dbufB)
        for j in range(BLK):
            scat_wait(j, dbufB)

            @pl.when(jnp.logical_not(last))
            def _(j=j):
                fire(j, sbufA)
        return 0

    lax.fori_loop(0, NBLK_MAIN // 2, outer, 0)
    plsc.subcore_barrier()
    pltpu.sync_copy(acc.at[pl.ds(s * ROWS_PER_TILE, ROWS_PER_TILE)],
                    out_hbm.at[pl.ds((c * NS + s) * ROWS_PER_TILE,
                                     ROWS_PER_TILE)])


_SC_CACHE = []


def _sc_calls():
    if _SC_CACHE:
        return _SC_CACHE[0]
    mesh = plsc.VectorSubcoreMesh(core_axis_name="c", subcore_axis_name="s",
                                  num_cores=NC, num_subcores=NS)
    params = pltpu.CompilerParams(use_tc_tiling_on_sc=False)
    deg = pl.kernel(
        _deg_body,
        out_type=jax.ShapeDtypeStruct((NC * N_ACC, 32), jnp.float32),
        mesh=mesh,
        compiler_params=params,
        scratch_types=[
            pltpu.VMEM((BLK, 128), jnp.int32),
            pltpu.VMEM((BLK, 128), jnp.int32),
            pltpu.VMEM((128, 32), jnp.float32),
            pltpu.VMEM((128, 32), jnp.float32),
            pltpu.VMEM_SHARED((N_ACC, 32), jnp.float32),
            pltpu.SemaphoreType.DMA,
            pltpu.SemaphoreType.DMA,
            pltpu.SemaphoreType.DMA,
            pltpu.SemaphoreType.DMA,
        ],
    )
    scat = pl.kernel(
        _scat_body,
        out_type=jax.ShapeDtypeStruct((NC * N_ACC, 32), jnp.float32),
        mesh=mesh,
        compiler_params=params,
        scratch_types=[
            pltpu.VMEM((BLK, 128), jnp.int32),
            pltpu.VMEM((BLK, 128), jnp.int32),
            pltpu.VMEM((BLK, 128), jnp.int32),
            pltpu.VMEM((BLK, 128), jnp.int32),
            pltpu.VMEM((BLK * 128, 32), jnp.float32),
            pltpu.VMEM((ZROWS, 32), jnp.float32),
            pltpu.VMEM_SHARED((N_ACC, 32), jnp.float32),
            pltpu.SemaphoreType.DMA,
            pltpu.SemaphoreType.DMA,
            pltpu.SemaphoreType.DMA,
            pltpu.SemaphoreType.DMA,
            pltpu.SemaphoreType.DMA,
            pltpu.SemaphoreType.DMA,
            pltpu.SemaphoreType.DMA,
            pltpu.SemaphoreType.DMA,
        ],
    )
    _SC_CACHE.append((deg, scat))
    return _SC_CACHE[0]


def _deg_call(dst_r):
    return _sc_calls()[0](dst_r)


def _scat_call(src2, dst_r, tab):
    return _sc_calls()[1](src2, dst_r, tab)


def _dinv_j(dlo_ref, dhi_ref, j):
    d = (dlo_ref[:, 32 * j:32 * j + 1] + dhi_ref[:, 32 * j:32 * j + 1] + 1.0)
    return lax.rsqrt(d)


def _d1_body(x_ref, w_ref, dlo_ref, dhi_ref, out_ref):
    for j in range(4):
        xj = x_ref[pl.ds(j * PR, PR), :]
        h = jnp.dot(xj, w_ref[0], preferred_element_type=jnp.float32)
        out_ref[:, 32 * j:32 * (j + 1)] = h * _dinv_j(dlo_ref, dhi_ref, j)


_d1_call = pl.pallas_call(
    _d1_body,
    grid=(2, NRB),
    in_specs=[
        pl.BlockSpec((RB, F_IN), lambda h, i: (i, 0)),
        pl.BlockSpec((1, F_IN, 32), lambda h, i: (h, 0, 0)),
        pl.BlockSpec((PR, 128), lambda h, i: (i, 0)),
        pl.BlockSpec((PR, 128), lambda h, i: (NBA + i, 0)),
    ],
    out_specs=pl.BlockSpec((PR, 128), lambda h, i: (h * NRB + i, 0)),
    out_shape=jax.ShapeDtypeStruct((2 * N_PAD // 4, 128), jnp.float32),
)


def _zpart_j(alo, ahi, hlo, hhi, dinv, b_ref, j):
    lo = (alo[:, 32 * j:32 * (j + 1)] + hlo[:, 32 * j:32 * (j + 1)])
    hi = (ahi[:, 32 * j:32 * (j + 1)] + hhi[:, 32 * j:32 * (j + 1)])
    return jnp.concatenate([lo, hi], axis=1) * dinv + b_ref[...]


def _stat_mask(k, j):
    p = lax.broadcasted_iota(jnp.int32, (PR, 1), 0)
    nid = (k * PR + p) * 4 + j
    return nid < N


def _phase1(k, acc_lo, acc_hi, hp_lo, hp_hi, dlo, dhi, b_ref, zbuf, sacc):
    alo, ahi = acc_lo[...], acc_hi[...]
    hlo, hhi = hp_lo[...], hp_hi[...]
    tot = None
    for j in range(4):
        dinv = _dinv_j(dlo, dhi, j)
        z = _zpart_j(alo, ahi, hlo, hhi, dinv, b_ref, j)
        zbuf[pl.ds(k * PR, PR), 64 * j:64 * (j + 1)] = z
        zm = jnp.where(_stat_mask(k, j), z, 0.0)
        part = jnp.concatenate([jnp.sum(zm, axis=0, keepdims=True),
                                jnp.sum(zm * zm, axis=0, keepdims=True)],
                               axis=0)
        tot = part if tot is None else tot + part

    @pl.when(k == 0)
    def _():
        sacc[...] = tot

    @pl.when(k > 0)
    def _():
        sacc[...] = sacc[...] + tot


def _bn_coefs(sacc, g_ref, be_ref):
    m = sacc[0:1, :] * (1.0 / N)
    var = sacc[1:2, :] * (1.0 / N) - m * m
    sc = g_ref[...] * lax.rsqrt(var + 1e-5)
    sh = be_ref[...] - m * sc
    return sc, sh


def _m_body(acc_lo, acc_hi, hp_lo, hp_hi, dlo, dhi, b_ref, g_ref, be_ref,
            w_ref, out_ref, zbuf, sacc):
    k = pl.program_id(0)

    @pl.when(k < NRB)
    def _():
        _phase1(k, acc_lo, acc_hi, hp_lo, hp_hi, dlo, dhi, b_ref, zbuf, sacc)

    @pl.when(k >= NRB)
    def _():
        i2 = lax.rem(k - NRB, NRB)
        sc, sh = _bn_coefs(sacc[...], g_ref, be_ref)
        for j in range(4):
            zj = zbuf[pl.ds(i2 * PR, PR), 64 * j:64 * (j + 1)]
            act = jnp.maximum(zj * sc + sh, 0.0)
            h = jnp.dot(act, w_ref[0], preferred_element_type=jnp.float32)
            out_ref[:, 32 * j:32 * (j + 1)] = h * _dinv_j(dlo, dhi, j)


def _zk(k):
    return jnp.where(k < NRB, k, lax.rem(k - NRB, NRB))


_m_call = pl.pallas_call(
    _m_body,
    grid=(3 * NRB,),
    in_specs=[
        pl.BlockSpec((PR, 128), lambda k: (jnp.where(k < NRB, k, 0), 0)),
        pl.BlockSpec((PR, 128),
                     lambda k: (NBA + jnp.where(k < NRB, k, 0), 0)),
        pl.BlockSpec((PR, 128), lambda k: (jnp.where(k < NRB, k, 0), 0)),
        pl.BlockSpec((PR, 128),
                     lambda k: (NRB + jnp.where(k < NRB, k, 0), 0)),
        pl.BlockSpec((PR, 128), lambda k: (_zk(k), 0)),
        pl.BlockSpec((PR, 128), lambda k: (NBA + _zk(k), 0)),
        pl.BlockSpec((1, H), lambda k: (0, 0)),
        pl.BlockSpec((1, H), lambda k: (0, 0)),
        pl.BlockSpec((1, H), lambda k: (0, 0)),
        pl.BlockSpec((1, H, 32),
                     lambda k: (jnp.where(k < NRB, 0, (k - NRB) // NRB),
                                0, 0)),
    ],
    out_specs=pl.BlockSpec(
        (PR, 128),
        lambda k: (jnp.where(k < NRB, 0, (k - NRB) // NRB) * NRB
                   + jnp.where(k < NRB, 0, lax.rem(k - NRB, NRB)), 0)),
    out_shape=jax.ShapeDtypeStruct((2 * N_PAD // 4, 128), jnp.float32),
    scratch_shapes=[pltpu.VMEM((N_PAD // 4, 256), jnp.float32),
                    pltpu.VMEM((2, H), jnp.float32)],
)


def _p_body(acc_lo, acc_hi, hp_lo, hp_hi, dlo, dhi, b_ref, g_ref, be_ref,
            bat_ref, gc_ref, mal_ref, t_ref, out_ref, zbuf, sacc, psum, pcnt):
    k = pl.program_id(0)

    @pl.when(k < NRB)
    def _():
        _phase1(k, acc_lo, acc_hi, hp_lo, hp_hi, dlo, dhi, b_ref, zbuf, sacc)

    @pl.when(k >= NRB)
    def _():
        i2 = k - NRB
        sc, sh = _bn_coefs(sacc[...], g_ref, be_ref)
        ps = None
        pc = None
        gids = lax.broadcasted_iota(jnp.int32, (PR, G), 1)
        for j in range(4):
            zj = zbuf[pl.ds(i2 * PR, PR), 64 * j:64 * (j + 1)]
            nz = zj * sc + sh
            bat = bat_ref[pl.ds(j * PR, PR), :]
            oneh = (bat == gids).astype(jnp.float32)
            psj = lax.dot_general(oneh, nz, (((0,), (0,)), ((), ())),
                                  preferred_element_type=jnp.float32)
            pcj = lax.dot_general(oneh, jnp.ones((PR, 8), jnp.float32),
                                  (((0,), (0,)), ((), ())),
                                  preferred_element_type=jnp.float32)
            ps = psj if ps is None else ps + psj
            pc = pcj if pc is None else pc + pcj

        @pl.when(i2 == 0)
        def _():
            psum[...] = ps
            pcnt[...] = pc

        @pl.when(i2 > 0)
        def _():
            psum[...] = psum[...] + ps
            pcnt[...] = pcnt[...] + pc

    @pl.when(k == 2 * NRB - 1)
    def _():
        emb = psum[...] / jnp.maximum(pcnt[...][:, 0:1], 1.0)
        e2 = jnp.sum(emb * emb, axis=1, keepdims=True)
        gcT = gc_ref[...]
        gd = (e2 + jnp.sum(gcT * gcT, axis=0, keepdims=True)
              - 2.0 * jnp.dot(emb, gcT, preferred_element_type=jnp.float32))
        ming = jnp.min(gd, axis=1, keepdims=True)
        mm = None
        for kk in range(CPF):
            mk = mal_ref[kk]
            mdk = (e2 + jnp.sum(mk * mk, axis=0, keepdims=True)
                   - 2.0 * jnp.dot(emb, mk,
                                   preferred_element_type=jnp.float32))
            mm = mdk if mm is None else jnp.minimum(mm, mdk)
        tv = t_ref[0, 0]
        out_ref[...] = jnp.concatenate([-ming, -mm], axis=1) / tv


_p_call = pl.pallas_call(
    _p_body,
    grid=(2 * NRB,),
    in_specs=[
        pl.BlockSpec((PR, 128), lambda k: (jnp.where(k < NRB, k, 0), 0)),
        pl.BlockSpec((PR, 128),
                     lambda k: (NBA + jnp.where(k < NRB, k, 0), 0)),
        pl.BlockSpec((PR, 128), lambda k: (jnp.where(k < NRB, k, 0), 0)),
        pl.BlockSpec((PR, 128),
                     lambda k: (NRB + jnp.where(k < NRB, k, 0), 0)),
        pl.BlockSpec((PR, 128), lambda k: (jnp.where(k < NRB, k, 0), 0)),
        pl.BlockSpec((PR, 128),
                     lambda k: (NBA + jnp.where(k < NRB, k, 0), 0)),
        pl.BlockSpec((1, H), lambda k: (0, 0)),
        pl.BlockSpec((1, H), lambda k: (0, 0)),
        pl.BlockSpec((1, H), lambda k: (0, 0)),
        pl.BlockSpec((RB, 1), lambda k: (jnp.where(k < NRB, 0, k - NRB), 0)),
        pl.BlockSpec((H, 8), lambda k: (0, 0)),
        pl.BlockSpec((CPF, H, NUM_FAM), lambda k: (0, 0, 0)),
        pl.BlockSpec((1, 1), lambda k: (0, 0)),
    ],
    out_specs=pl.BlockSpec((G, 1 + NUM_FAM), lambda k: (0, 0)),
    out_shape=jax.ShapeDtypeStruct((G, 1 + NUM_FAM), jnp.float32),
    scratch_shapes=[pltpu.VMEM((N_PAD // 4, 256), jnp.float32),
                    pltpu.VMEM((2, H), jnp.float32),
                    pltpu.VMEM((G, H), jnp.float32),
                    pltpu.VMEM((G, 8), jnp.float32)],
)


def kernel(x, edge_index, batch, W1, b1, W2, b2, W3, b3, g1, be1, g2, be2,
           g3, be3, mal_c, good_c, temp):
    f32 = jnp.float32
    src = edge_index[0]
    dst = edge_index[1]
    pad = jnp.arange(E_PAD - E, dtype=jnp.int32)
    src_p = jnp.concatenate([src, pad % N])
    dst_p = jnp.concatenate([dst, N + pad % (N_ACC - N)])
    src2 = jnp.concatenate([src_p, src_p + N_PAD]).reshape(
        2 * E_PAD // 128, 128)
    dst_r = dst_p.reshape(E_PAD // 128, 128)

    x_pad = jnp.concatenate([x, jnp.zeros((N_PAD - N, F_IN), f32)])
    x_perm = x_pad.reshape(NRB, PR, 4, F_IN).transpose(0, 2, 1, 3)
    x_perm = x_perm.reshape(N_PAD, F_IN)
    bat_pad = jnp.concatenate(
        [batch, jnp.full((N_PAD - N,), G, jnp.int32)])
    bat_perm = bat_pad.reshape(NRB, PR, 4).transpose(0, 2, 1)
    bat_perm = bat_perm.reshape(N_PAD, 1)

    w1s = W1.reshape(F_IN, 2, 32).transpose(1, 0, 2)
    w2s = W2.reshape(H, 2, 32).transpose(1, 0, 2)
    w3s = W3.reshape(H, 2, 32).transpose(1, 0, 2)

    degp = _deg_call(dst_r).reshape(NC * N_ACC // 4, 128)

    hp1 = _d1_call(x_perm, w1s, degp, degp)
    acc1 = _scat_call(src2, dst_r, hp1.reshape(2 * N_PAD, 32))
    acc1 = acc1.reshape(NC * N_ACC // 4, 128)
    hp2 = _m_call(acc1, acc1, hp1, hp1, degp, degp, b1.reshape(1, H),
                  g1.reshape(1, H), be1.reshape(1, H), w2s)
    acc2 = _scat_call(src2, dst_r, hp2.reshape(2 * N_PAD, 32))
    acc2 = acc2.reshape(NC * N_ACC // 4, 128)
    hp3 = _m_call(acc2, acc2, hp2, hp2, degp, degp, b2.reshape(1, H),
                  g2.reshape(1, H), be2.reshape(1, H), w3s)
    acc3 = _scat_call(src2, dst_r, hp3.reshape(2 * N_PAD, 32))
    acc3 = acc3.reshape(NC * N_ACC // 4, 128)

    good_pad = jnp.concatenate([good_c, jnp.full((3, H), 1e4, f32)], axis=0)
    mal3 = mal_c.reshape(NUM_FAM, CPF, H).transpose(1, 2, 0)
    logits = _p_call(acc3, acc3, hp3, hp3, degp, degp, b3.reshape(1, H),
                     g3.reshape(1, H), be3.reshape(1, H), bat_perm,
                     good_pad.T, mal3, temp.reshape(1, 1))
    return logits
